# baseline (speedup 1.0000x reference)
import functools

import jax
import jax.numpy as jnp
import numpy as np
from jax import lax
from jax.experimental import pallas as pl
from jax.experimental.pallas import tpu as pltpu
from jax.experimental.pallas import tpu_sc as plsc

_NC, _NS, _L = 2, 16, 16
_W = 2048
_SPMEM_BYTES = 6_500_000


def _prep_edges(ei):
    e = ei.shape[1]
    et = -(-e // (_NS * _W)) * _W
    e_pad = et * _NS
    pad = e_pad - e
    pad_src = (jnp.arange(pad, dtype=jnp.int32) % 256)
    src = jnp.concatenate([ei[0].astype(jnp.int32), pad_src])
    dst = jnp.concatenate([ei[1].astype(jnp.int32),
                           jnp.full((pad,), 1 << 29, jnp.int32)])
    return src.reshape(-1, _L), dst.reshape(-1, _L), et


def _chunking(n_out, d, tile_words=12000):
    budget_words = 1_950_000 - 16 * tile_words
    rmax = max(128, (budget_words // d - 64) // 128 * 128)
    n_chunks = -(-n_out // rmax)
    if n_chunks % 2:
        n_chunks += 1
    r = -(-n_out // (n_chunks * 128)) * 128
    return r, n_chunks


def _seg_sum_sc(x, src2, dst2, et, n_out):
    n_src, d = x.shape
    fb = 32 if d >= 128 else 64
    cap = _W + 160
    tile_words = 2 * (fb * d + 2 * fb) + 2 * 2048 + 2 * cap + 3000
    r, n_chunks = _chunking(n_out, d, tile_words)
    n_pad = r * n_chunks
    nw = et // _W
    trows = et // _L
    wrows = _W // _L
    nzb = (r + 64) // 64
    ncb = r // 64
    mesh = plsc.VectorSubcoreMesh(core_axis_name="c", subcore_axis_name="s",
                                  num_cores=_NC)

    @functools.partial(
        pl.kernel, mesh=mesh,
        compiler_params=pltpu.CompilerParams(needs_layout_passes=False, use_tc_tiling_on_sc=False),
        out_type=jax.ShapeDtypeStruct((n_pad, d), jnp.float32),
        scratch_types=[
            pltpu.VMEM((wrows, _L), jnp.int32),
            pltpu.VMEM((wrows, _L), jnp.int32),
            pltpu.VMEM((cap,), jnp.int32),
            pltpu.VMEM((cap,), jnp.int32),
            pltpu.VMEM((fb,), jnp.int32),
            pltpu.VMEM((fb,), jnp.int32),
            pltpu.VMEM((fb, d), jnp.float32),
            pltpu.VMEM((fb,), jnp.int32),
            pltpu.VMEM((fb,), jnp.int32),
            pltpu.VMEM((fb, d), jnp.float32),
            pltpu.VMEM_SHARED((r + 64, d), jnp.float32),
            pltpu.SemaphoreType.DMA,
            pltpu.SemaphoreType.DMA,
            pltpu.SemaphoreType.DMA,
            pltpu.SemaphoreType.DMA,
        ],
    )
    def k(x_hbm, src_hbm, dst_hbm, zeros_hbm, out_hbm,
          srcw, dstw, sstage, dstage, g_idx_a, sc_idx_a, rows_a,
          g_idx_b, sc_idx_b, rows_b, acc, sem1, sem2, sem3, sem4):
        bufs = ((g_idx_a, sc_idx_a, rows_a, sem1, sem3),
                (g_idx_b, sc_idx_b, rows_b, sem2, sem4))
        c = lax.axis_index("c")
        s = lax.axis_index("s")
        lane = lax.iota(jnp.int32, _L)
        pad_src = (s * _L + lane) % n_src

        for ci in range(n_chunks // 2):
            chunk = c * (n_chunks // 2) + ci
            lo = chunk * r
            lo_v = jnp.full((_L,), 0, jnp.int32) + lo
            hi_v = lo_v + r

            for jj in range((nzb + _NS - 1) // _NS):
                j = jj * _NS + s

                @pl.when(j < nzb)
                def _zero():
                    pltpu.sync_copy(zeros_hbm, acc.at[pl.ds(j * 64, 64), :])
            plsc.subcore_barrier()

            def window(w, _):
                wr0 = s * trows + w * wrows
                pltpu.sync_copy(src_hbm.at[pl.ds(wr0, wrows), :], srcw)
                pltpu.sync_copy(dst_hbm.at[pl.ds(wr0, wrows), :], dstw)

                def scan(b, cnt):
                    dvec = dstw[b, :]
                    svec = srcw[b, :]
                    m = (dvec >= lo_v) & (dvec < hi_v)
                    plsc.store_compressed(sstage.at[pl.ds(cnt, _L)], svec, mask=m)
                    plsc.store_compressed(dstage.at[pl.ds(cnt, _L)],
                                          dvec - lo_v, mask=m)
                    return cnt + plsc.all_reduce_population_count(m)[0]
                cnt = lax.fori_loop(0, wrows, scan, jnp.int32(0))

                trash = jnp.full((_L,), r, jnp.int32)
                for kk in range(fb // _L):
                    sstage[pl.ds(cnt + kk * _L, _L)] = pad_src
                    dstage[pl.ds(cnt + kk * _L, _L)] = trash

                nb = (cnt + fb - 1) // fb

                def drain_scatter(half):
                    gi, sci, rws, gsem, ssem = bufs[half]
                    pltpu.make_async_copy(rws, acc.at[sci], ssem).wait()

                def fetch(j, half):
                    gi, sci, rws, gsem, ssem = bufs[half]

                    @pl.when((j >= 2) & (j < nb))
                    def _dr():
                        drain_scatter(half)

                    @pl.when(j < nb)
                    def _():
                        for kk in range(fb // _L):
                            gi[pl.ds(kk * _L, _L)] = (
                                sstage[pl.ds(j * fb + kk * _L, _L)])
                            sci[pl.ds(kk * _L, _L)] = (
                                dstage[pl.ds(j * fb + kk * _L, _L)])
                        pltpu.async_copy(x_hbm.at[gi], rws, gsem)

                def process(j, half):
                    gi, sci, rws, gsem, ssem = bufs[half]

                    @pl.when(j < nb)
                    def _():
                        pltpu.make_async_copy(x_hbm.at[gi], rws, gsem).wait()
                        pltpu.async_copy(rws, acc.at[sci], ssem, add=True)

                fetch(0, 0)
                fetch(1, 1)

                def piped(jj, _):
                    j0 = jj * 2
                    process(j0, 0)
                    process(j0 + 1, 1)
                    fetch(j0 + 2, 0)
                    fetch(j0 + 3, 1)
                    return 0
                lax.fori_loop(0, (nb + 1) // 2, piped, 0)

                @pl.when(nb >= 1)
                def _d0():
                    drain_scatter(0)

                @pl.when(nb >= 2)
                def _d1():
                    drain_scatter(1)
                return 0
            lax.fori_loop(0, nw, window, 0)
            plsc.subcore_barrier()

            for jj in range((ncb + _NS - 1) // _NS):
                j = jj * _NS + s

                @pl.when(j < ncb)
                def _out():
                    pltpu.sync_copy(acc.at[pl.ds(j * 64, 64), :],
                                    out_hbm.at[pl.ds(lo + j * 64, 64), :])
            plsc.subcore_barrier()

    zeros = jnp.zeros((64, d), jnp.float32)
    return k(x, src2, dst2, zeros)


def _degree_sc(dst2, et, n_out):
    r, n_chunks = _chunking(n_out, 8)
    n_pad = r * n_chunks
    nw = et // _W
    trows = et // _L
    wrows = _W // _L
    nzb = (r + 64) // 64
    ncb = r // 64
    mesh = plsc.VectorSubcoreMesh(core_axis_name="c", subcore_axis_name="s",
                                  num_cores=_NC)

    @functools.partial(
        pl.kernel, mesh=mesh,
        compiler_params=pltpu.CompilerParams(needs_layout_passes=False, use_tc_tiling_on_sc=False),
        out_type=jax.ShapeDtypeStruct((n_pad, 8), jnp.float32),
        scratch_types=[
            pltpu.VMEM((wrows, _L), jnp.int32),
            pltpu.VMEM((128,), jnp.int32),
            pltpu.VMEM((128, 8), jnp.float32),
            pltpu.VMEM_SHARED((r + 64, 8), jnp.float32),
        ],
    )
    def k(dst_hbm, ones_hbm, zeros_hbm, out_hbm, dstw, sc_idx, ones, acc):
        c = lax.axis_index("c")
        s = lax.axis_index("s")
        pltpu.sync_copy(ones_hbm, ones)

        for ci in range(n_chunks // 2):
            chunk = c * (n_chunks // 2) + ci
            lo = chunk * r
            lo_v = jnp.full((_L,), 0, jnp.int32) + lo
            hi_v = lo_v + r

            for jj in range((nzb + _NS - 1) // _NS):
                j = jj * _NS + s

                @pl.when(j < nzb)
                def _zero():
                    pltpu.sync_copy(zeros_hbm, acc.at[pl.ds(j * 64, 64), :])
            plsc.subcore_barrier()

            def window(w, _):
                wr0 = s * trows + w * wrows
                pltpu.sync_copy(dst_hbm.at[pl.ds(wr0, wrows), :], dstw)

                def batch(j, _):
                    trash = jnp.full((_L,), r, jnp.int32)
                    for kk in range(8):
                        dvec = dstw[j * 8 + kk, :]
                        m = (dvec >= lo_v) & (dvec < hi_v)
                        sc_idx[pl.ds(kk * _L, _L)] = jnp.where(
                            m, dvec - lo_v, trash)
                    pltpu.sync_copy(ones, acc.at[sc_idx], add=True)
                    return 0
                lax.fori_loop(0, wrows // 8, batch, 0)
                return 0
            lax.fori_loop(0, nw, window, 0)
            plsc.subcore_barrier()

            for jj in range((ncb + _NS - 1) // _NS):
                j = jj * _NS + s

                @pl.when(j < ncb)
                def _out():
                    pltpu.sync_copy(acc.at[pl.ds(j * 64, 64), :],
                                    out_hbm.at[pl.ds(lo + j * 64, 64), :])
            plsc.subcore_barrier()

    ones = jnp.ones((128, 8), jnp.float32)
    zeros = jnp.zeros((64, 8), jnp.float32)
    return k(dst2, ones, zeros)


def _edge_softmax_sc(a, src2, dst2, et, n_out, cvec):
    n = a.shape[0]
    heads = 4
    r, n_chunks = _chunking(n_out, 8)
    n_pad = r * n_chunks
    nw = et // _W
    trows = et // _L
    wrows = _W // _L
    e_pad = et * _NS
    nzb = (r + 64) // 64
    ncb = r // 64
    mesh = plsc.VectorSubcoreMesh(core_axis_name="c", subcore_axis_name="s",
                                  num_cores=_NC)

    @functools.partial(
        pl.kernel, mesh=mesh,
        compiler_params=pltpu.CompilerParams(needs_layout_passes=False, use_tc_tiling_on_sc=False),
        out_type=[jax.ShapeDtypeStruct((e_pad, 8), jnp.float32),
                  jax.ShapeDtypeStruct((n_pad, 8), jnp.float32)],
        scratch_types=[
            pltpu.VMEM((wrows, _L), jnp.int32),
            pltpu.VMEM((wrows, _L), jnp.int32),
            pltpu.VMEM((128,), jnp.int32),
            pltpu.VMEM((128,), jnp.int32),
            pltpu.VMEM((128,), jnp.int32),
            pltpu.VMEM((128, 8), jnp.float32),
            pltpu.VMEM((128, 8), jnp.float32),
            pltpu.VMEM((128, 8), jnp.float32),
            pltpu.VMEM((_L,), jnp.float32),
            pltpu.VMEM_SHARED((r + 64, 8), jnp.float32),
            pltpu.SemaphoreType.DMA,
            pltpu.SemaphoreType.DMA,
        ],
    )
    def k(a_hbm, src_hbm, dst_hbm, c_hbm, zeros_hbm, ex_hbm, den_hbm,
          srcw, dstw, g1_idx, g2_idx, sc_idx, arows_s, arows_d, exw, cbuf,
          acc, sem1, sem2):
        c = lax.axis_index("c")
        s = lax.axis_index("s")
        lane = lax.iota(jnp.int32, _L)
        row4b = lane // heads
        colS = lane % heads
        pltpu.sync_copy(c_hbm, cbuf)
        cv = cbuf[...]

        chunk = c
        lo = chunk * r
        lo_v = jnp.full((_L,), 0, jnp.int32) + lo
        hi_v = lo_v + r
        trash = jnp.full((_L,), r, jnp.int32)

        for jj in range((nzb + _NS - 1) // _NS):
            j = jj * _NS + s

            @pl.when(j < nzb)
            def _zero():
                pltpu.sync_copy(zeros_hbm, acc.at[pl.ds(j * 64, 64), :])
        plsc.subcore_barrier()

        def window(w, _):
            wr0 = s * trows + w * wrows
            pltpu.sync_copy(src_hbm.at[pl.ds(wr0, wrows), :], srcw)
            pltpu.sync_copy(dst_hbm.at[pl.ds(wr0, wrows), :], dstw)

            def batch(j, _):
                nmax = jnp.full((_L,), n - 1, jnp.int32)
                for kk in range(8):
                    g1_idx[pl.ds(kk * _L, _L)] = jnp.minimum(
                        srcw[j * 8 + kk, :], nmax)
                    g2_idx[pl.ds(kk * _L, _L)] = jnp.minimum(
                        dstw[j * 8 + kk, :], nmax)
                cp1 = pltpu.async_copy(a_hbm.at[g1_idx], arows_s, sem1)
                cp2 = pltpu.async_copy(a_hbm.at[g2_idx], arows_d, sem2)
                cp1.wait()
                cp2.wait()

                def grp(b2, _):
                    row = b2 * 4 + row4b
                    vs = plsc.load_gather(arows_s, [row, colS])
                    vd = plsc.load_gather(arows_d, [row, colS + heads])
                    v = vs + vd
                    v = jnp.where(v >= 0.0, v, 0.2 * v) - cv
                    plsc.store_scatter(exw, [row, colS], jnp.exp(v))
                    return 0
                lax.fori_loop(0, 32, grp, 0)

                for kk in range(8):
                    dvec = dstw[j * 8 + kk, :]
                    m = (dvec >= lo_v) & (dvec < hi_v)
                    sc_idx[pl.ds(kk * _L, _L)] = jnp.where(
                        m, dvec - lo_v, trash)
                pltpu.sync_copy(exw, acc.at[sc_idx], add=True)

                @pl.when(c == 0)
                def _wr_ex():
                    pltpu.sync_copy(
                        exw, ex_hbm.at[pl.ds(s * et + w * _W + j * 128,
                                             128), :])
                return 0
            lax.fori_loop(0, wrows // 8, batch, 0)
            return 0
        lax.fori_loop(0, nw, window, 0)
        plsc.subcore_barrier()

        for jj in range((ncb + _NS - 1) // _NS):
            j = jj * _NS + s

            @pl.when(j < ncb)
            def _out():
                pltpu.sync_copy(acc.at[pl.ds(j * 64, 64), :],
                                den_hbm.at[pl.ds(lo + j * 64, 64), :])
        plsc.subcore_barrier()

    zeros = jnp.zeros((64, 8), jnp.float32)
    return k(a, src2, dst2, cvec, zeros)


def _weighted_seg_sum_sc(xw, ex, src2, dst2, et, n_out):
    n_src, d = xw.shape
    heads = 4
    o = d // heads
    fb = 32
    cap = _W + 160
    tile_words = 2 * (fb * d + fb * 8 + 3 * fb) + 2 * 2048 + 3 * cap + 3000
    r, n_chunks = _chunking(n_out, d, tile_words)
    n_pad = r * n_chunks
    nw = et // _W
    trows = et // _L
    wrows = _W // _L
    nzb = (r + 64) // 64
    ncb = r // 64
    mesh = plsc.VectorSubcoreMesh(core_axis_name="c", subcore_axis_name="s",
                                  num_cores=_NC)

    @functools.partial(
        pl.kernel, mesh=mesh,
        compiler_params=pltpu.CompilerParams(needs_layout_passes=False, use_tc_tiling_on_sc=False),
        out_type=jax.ShapeDtypeStruct((n_pad, d), jnp.float32),
        scratch_types=[
            pltpu.VMEM((wrows, _L), jnp.int32),
            pltpu.VMEM((wrows, _L), jnp.int32),
            pltpu.VMEM((cap,), jnp.int32),
            pltpu.VMEM((cap,), jnp.int32),
            pltpu.VMEM((cap,), jnp.int32),
            pltpu.VMEM((fb,), jnp.int32),
            pltpu.VMEM((fb,), jnp.int32),
            pltpu.VMEM((fb,), jnp.int32),
            pltpu.VMEM((fb, d), jnp.float32),
            pltpu.VMEM((fb, 8), jnp.float32),
            pltpu.VMEM((fb,), jnp.int32),
            pltpu.VMEM((fb,), jnp.int32),
            pltpu.VMEM((fb,), jnp.int32),
            pltpu.VMEM((fb, d), jnp.float32),
            pltpu.VMEM((fb, 8), jnp.float32),
            pltpu.VMEM_SHARED((r + 64, d), jnp.float32),
            pltpu.SemaphoreType.DMA,
            pltpu.SemaphoreType.DMA,
            pltpu.SemaphoreType.DMA,
            pltpu.SemaphoreType.DMA,
        ],
    )
    def k(xw_hbm, ex_hbm, src_hbm, dst_hbm, zeros_hbm, out_hbm,
          srcw, dstw, sstage, dstage, estage, g_idx_a, e_idx_a, sc_idx_a,
          rows_a, exr_a, g_idx_b, e_idx_b, sc_idx_b, rows_b, exr_b,
          acc, sem1, sem2, sem3, sem4):
        bufs = ((g_idx_a, e_idx_a, sc_idx_a, rows_a, exr_a, sem1, sem3),
                (g_idx_b, e_idx_b, sc_idx_b, rows_b, exr_b, sem2, sem4))
        c = lax.axis_index("c")
        s = lax.axis_index("s")
        lane = lax.iota(jnp.int32, _L)
        pad_src = (s * _L + lane) % n_src
        pad_eid = s * _L + lane

        for ci in range(n_chunks // 2):
            chunk = c * (n_chunks // 2) + ci
            lo = chunk * r
            lo_v = jnp.full((_L,), 0, jnp.int32) + lo
            hi_v = lo_v + r
            trash = jnp.full((_L,), r, jnp.int32)

            for jj in range((nzb + _NS - 1) // _NS):
                j = jj * _NS + s

                @pl.when(j < nzb)
                def _zero():
                    pltpu.sync_copy(zeros_hbm, acc.at[pl.ds(j * 64, 64), :])
            plsc.subcore_barrier()

            def window(w, _):
                wr0 = s * trows + w * wrows
                pltpu.sync_copy(src_hbm.at[pl.ds(wr0, wrows), :], srcw)
                pltpu.sync_copy(dst_hbm.at[pl.ds(wr0, wrows), :], dstw)
                ebase = s * et + w * _W

                def scan(b, cnt):
                    dvec = dstw[b, :]
                    svec = srcw[b, :]
                    evec = ebase + b * _L + lane
                    m = (dvec >= lo_v) & (dvec < hi_v)
                    plsc.store_compressed(sstage.at[pl.ds(cnt, _L)], svec, mask=m)
                    plsc.store_compressed(dstage.at[pl.ds(cnt, _L)],
                                          dvec - lo_v, mask=m)
                    plsc.store_compressed(estage.at[pl.ds(cnt, _L)], evec, mask=m)
                    return cnt + plsc.all_reduce_population_count(m)[0]
                cnt = lax.fori_loop(0, wrows, scan, jnp.int32(0))

                for kk in range(fb // _L):
                    sstage[pl.ds(cnt + kk * _L, _L)] = pad_src
                    dstage[pl.ds(cnt + kk * _L, _L)] = trash
                    estage[pl.ds(cnt + kk * _L, _L)] = pad_eid

                nb = (cnt + fb - 1) // fb

                def drain_scatter(half):
                    gi, eix, sci, rws, exv, gsem, ssem = bufs[half]
                    pltpu.make_async_copy(rws, acc.at[sci], ssem).wait()

                def fetch(j, half):
                    gi, eix, sci, rws, exv, gsem, ssem = bufs[half]

                    @pl.when((j >= 2) & (j < nb))
                    def _dr():
                        drain_scatter(half)

                    @pl.when(j < nb)
                    def _():
                        for kk in range(fb // _L):
                            gi[pl.ds(kk * _L, _L)] = (
                                sstage[pl.ds(j * fb + kk * _L, _L)])
                            sci[pl.ds(kk * _L, _L)] = (
                                dstage[pl.ds(j * fb + kk * _L, _L)])
                            eix[pl.ds(kk * _L, _L)] = (
                                estage[pl.ds(j * fb + kk * _L, _L)])
                        pltpu.async_copy(xw_hbm.at[gi], rws, gsem)
                        pltpu.async_copy(ex_hbm.at[eix], exv, gsem)

                def process(j, half):
                    gi, eix, sci, rws, exv, gsem, ssem = bufs[half]

                    @pl.when(j < nb)
                    def _():
                        pltpu.make_async_copy(xw_hbm.at[gi], rws, gsem).wait()
                        pltpu.make_async_copy(ex_hbm.at[eix], exv, gsem).wait()

                        def scale(rr, _):
                            for h in range(heads):
                                mult = plsc.load_gather(
                                    exv,
                                    [jnp.full((_L,), 0, jnp.int32) + rr,
                                     jnp.full((_L,), h, jnp.int32)])
                                for g in range(o // _L):
                                    c0 = h * o + g * _L
                                    rws[rr, pl.ds(c0, _L)] = (
                                        rws[rr, pl.ds(c0, _L)] * mult)
                            return 0
                        lax.fori_loop(0, fb, scale, 0)
                        pltpu.async_copy(rws, acc.at[sci], ssem, add=True)

                fetch(0, 0)
                fetch(1, 1)

                def piped(jj, _):
                    j0 = jj * 2
                    process(j0, 0)
                    process(j0 + 1, 1)
                    fetch(j0 + 2, 0)
                    fetch(j0 + 3, 1)
                    return 0
                lax.fori_loop(0, (nb + 1) // 2, piped, 0)

                @pl.when(nb >= 1)
                def _d0():
                    drain_scatter(0)

                @pl.when(nb >= 2)
                def _d1():
                    drain_scatter(1)
                return 0
            lax.fori_loop(0, nw, window, 0)
            plsc.subcore_barrier()

            for jj in range((ncb + _NS - 1) // _NS):
                j = jj * _NS + s

                @pl.when(j < ncb)
                def _out():
                    pltpu.sync_copy(acc.at[pl.ds(j * 64, 64), :],
                                    out_hbm.at[pl.ds(lo + j * 64, 64), :])
            plsc.subcore_barrier()

    zeros = jnp.zeros((64, d), jnp.float32)
    return k(xw, ex, src2, dst2, zeros)


def _mm_body(x_ref, w_ref, b_ref, o_ref, *, act):
    h = jnp.dot(x_ref[...], w_ref[...], preferred_element_type=jnp.float32)
    h = h + b_ref[...]
    if act == "relu":
        h = jnp.maximum(h, 0.0)
    elif act == "elu":
        h = jnp.where(h > 0.0, h, jnp.exp(h) - 1.0)
    o_ref[...] = h


def _pick_bn(n):
    for bn in (2000, 1000, 500, 200, 100, 50, 25, 10, 8, 5, 4, 2, 1):
        if n % bn == 0:
            return bn
    return n


def _mm(x, w, b=None, act=None):
    n, k = x.shape
    o = w.shape[1]
    if b is None:
        b = jnp.zeros((o,), jnp.float32)
    bn = _pick_bn(n)
    return pl.pallas_call(
        functools.partial(_mm_body, act=act),
        grid=(n // bn,),
        in_specs=[
            pl.BlockSpec((bn, k), lambda i: (i, 0)),
            pl.BlockSpec((k, o), lambda i: (0, 0)),
            pl.BlockSpec((1, o), lambda i: (0, 0)),
        ],
        out_specs=pl.BlockSpec((bn, o), lambda i: (i, 0)),
        out_shape=jax.ShapeDtypeStruct((n, o), jnp.float32),
    )(x, w, b.reshape(1, o))


def _pool_body(x_ref, s_ref, m_ref):
    i = pl.program_id(0)
    blk = x_ref[...]
    bs = jnp.sum(blk, axis=0, keepdims=True)
    bm = jnp.max(blk, axis=0, keepdims=True)

    @pl.when(i == 0)
    def _init():
        s_ref[...] = bs
        m_ref[...] = bm

    @pl.when(i > 0)
    def _acc():
        s_ref[...] = s_ref[...] + bs
        m_ref[...] = jnp.maximum(m_ref[...], bm)


def _pool_sum_max(x):
    n, d = x.shape
    bn = _pick_bn(n)
    return pl.pallas_call(
        _pool_body,
        grid=(n // bn,),
        in_specs=[pl.BlockSpec((bn, d), lambda i: (i, 0))],
        out_specs=[pl.BlockSpec((1, d), lambda i: (0, 0)),
                   pl.BlockSpec((1, d), lambda i: (0, 0))],
        out_shape=[jax.ShapeDtypeStruct((1, d), jnp.float32),
                   jax.ShapeDtypeStruct((1, d), jnp.float32)],
    )(x)


def _seg_sum(vals, idx, n):
    return jax.ops.segment_sum(vals, idx, num_segments=n)


def _gin_mm_body(h_ref, agg_ref, eps_ref, w_ref, b_ref, o_ref):
    z = (1.0 + eps_ref[0, 0]) * h_ref[...] + agg_ref[...]
    z = jnp.dot(z, w_ref[...], preferred_element_type=jnp.float32)
    o_ref[...] = jnp.maximum(z + b_ref[...], 0.0)


def _gin_mm(h, agg_pad, eps, w, b):
    n, k = h.shape
    o = w.shape[1]
    bn = _pick_bn(n)
    return pl.pallas_call(
        _gin_mm_body,
        grid=(n // bn,),
        in_specs=[
            pl.BlockSpec((bn, k), lambda i: (i, 0)),
            pl.BlockSpec((bn, k), lambda i: (i, 0)),
            pl.BlockSpec((1, 1), lambda i: (0, 0), memory_space=pltpu.SMEM),
            pl.BlockSpec((k, o), lambda i: (0, 0)),
            pl.BlockSpec((1, o), lambda i: (0, 0)),
        ],
        out_specs=pl.BlockSpec((bn, o), lambda i: (i, 0)),
        out_shape=jax.ShapeDtypeStruct((n, o), jnp.float32),
    )(h, agg_pad, eps.reshape(1, 1), w, b.reshape(1, o))


def _sage_mm_body(s_ref, deg_ref, x_ref, wl_ref, wr_ref, b_ref, o_ref, *, act):
    inv = 1.0 / jnp.maximum(deg_ref[...][:, 0:1], 1.0)
    mean = s_ref[...] * inv
    z = (jnp.dot(mean, wl_ref[...], preferred_element_type=jnp.float32)
         + jnp.dot(x_ref[...], wr_ref[...], preferred_element_type=jnp.float32)
         + b_ref[...])
    if act == "relu":
        z = jnp.maximum(z, 0.0)
    o_ref[...] = z


def _sage_mm(s_pad, deg_pad, x, wl, wr, b, act):
    n, k = x.shape
    o = wl.shape[1]
    bn = _pick_bn(n)
    return pl.pallas_call(
        functools.partial(_sage_mm_body, act=act),
        grid=(n // bn,),
        in_specs=[
            pl.BlockSpec((bn, k), lambda i: (i, 0)),
            pl.BlockSpec((bn, 8), lambda i: (i, 0)),
            pl.BlockSpec((bn, k), lambda i: (i, 0)),
            pl.BlockSpec((k, o), lambda i: (0, 0)),
            pl.BlockSpec((k, o), lambda i: (0, 0)),
            pl.BlockSpec((1, o), lambda i: (0, 0)),
        ],
        out_specs=pl.BlockSpec((bn, o), lambda i: (i, 0)),
        out_shape=jax.ShapeDtypeStruct((n, o), jnp.float32),
    )(s_pad, deg_pad, x, wl, wr, b.reshape(1, o))


def _gat_fin_body(num_ref, xw_ref, a_ref, den_ref, c_ref, b_ref, o_ref,
                  *, heads, o, concat, act):
    bn = num_ref.shape[0]
    al = a_ref[...]
    e0 = al[:, :heads] + al[:, heads:2 * heads]
    e0 = jnp.where(e0 >= 0.0, e0, 0.2 * e0)
    el = jnp.exp(e0 - c_ref[0])
    den = den_ref[...][:, :heads] + el
    xw3 = xw_ref[...].reshape(bn, heads, o)
    num3 = num_ref[...].reshape(bn, heads, o) + el[:, :, None] * xw3
    out3 = num3 / den[:, :, None]
    if concat:
        z = out3.reshape(bn, heads * o) + b_ref[...]
    else:
        z = jnp.mean(out3, axis=1) + b_ref[...]
    if act == "elu":
        z = jnp.where(z > 0.0, z, jnp.exp(z) - 1.0)
    o_ref[...] = z


def _gat_finalize(num_pad, xw, a, den_pad, c, bias, concat, act):
    n, d = xw.shape
    heads = 4
    o = d // heads
    od = d if concat else o
    bn = _pick_bn(n)
    return pl.pallas_call(
        functools.partial(_gat_fin_body, heads=heads, o=o, concat=concat,
                          act=act),
        grid=(n // bn,),
        in_specs=[
            pl.BlockSpec((bn, d), lambda i: (i, 0)),
            pl.BlockSpec((bn, d), lambda i: (i, 0)),
            pl.BlockSpec((bn, 2 * heads), lambda i: (i, 0)),
            pl.BlockSpec((bn, 8), lambda i: (i, 0)),
            pl.BlockSpec(memory_space=pltpu.SMEM),
            pl.BlockSpec((1, od), lambda i: (0, 0)),
        ],
        out_specs=pl.BlockSpec((bn, od), lambda i: (i, 0)),
        out_shape=jax.ShapeDtypeStruct((n, od), jnp.float32),
    )(num_pad, xw, a, den_pad, c.reshape(1), bias.reshape(1, od))


def _gat_layer(p, h, src2, dst2, et, n, heads, o, concat, act):
    xw = _mm(h, p["W"])
    eye = jnp.eye(heads, dtype=jnp.float32)
    a_src = jnp.einsum("ho,hg->hog", p["att_src"], eye).reshape(heads * o, heads)
    a_dst = jnp.einsum("ho,hg->hog", p["att_dst"], eye).reshape(heads * o, heads)
    ab = jnp.concatenate([a_src, a_dst], axis=1)
    a = _mm(xw, ab)
    _, amax = _pool_sum_max(a)
    c = jnp.maximum(jnp.max(amax[0, :heads]) + jnp.max(amax[0, heads:]), 0.0)
    ex, den = _edge_softmax_sc(a, src2, dst2, et, n,
                               jnp.full((_L,), 1.0, jnp.float32) * c)
    num = _weighted_seg_sum_sc(xw, ex, src2, dst2, et, n)
    return _gat_finalize(num, xw, a, den, c, p["bias"], concat, act)


def _module_enc(p, x, ei):
    n = x.shape[0]
    src2, dst2, et = _prep_edges(ei)
    h = _mm(x, p["proj"]["w"], p["proj"]["b"], act="relu")
    h = _gat_layer(p["gat0"], h, src2, dst2, et, n, 4, 128, True, "elu")
    h = _gat_layer(p["gat1"], h, src2, dst2, et, n, 4, 128, True, "elu")
    h = _gat_layer(p["gat2"], h, src2, dst2, et, n, 4, 128, False, None)
    s, m = _pool_sum_max(h)
    hg = (s / n + m) / 2.0
    return _mm(hg, p["out"]["w"], p["out"]["b"])


def _dispatch_enc(p, x, ei):
    n = x.shape[0]
    src2, dst2, et = _prep_edges(ei)
    h = _mm(x, p["proj"]["w"], p["proj"]["b"], act="relu")
    for i in range(4):
        g = p["gin%d" % i]
        agg = _seg_sum_sc(h, src2, dst2, et, n)
        z = _gin_mm(h, agg, g["eps"], g["l1"]["w"], g["l1"]["b"])
        h = _mm(z, g["l2"]["w"], g["l2"]["b"], act="relu")
    s, _ = _pool_sum_max(h)
    hg = s / n
    hg = _mm(hg, p["out1"]["w"], p["out1"]["b"], act="relu")
    return _mm(hg, p["out2"]["w"], p["out2"]["b"])


def _type_enc(p, x, ei):
    n = x.shape[0]
    src2, dst2, et = _prep_edges(ei)
    deg = _degree_sc(dst2, et, n)
    h = x
    for name, act in (("sage0", "relu"), ("sage1", "relu"), ("sage2", None)):
        q = p[name]
        s = _seg_sum_sc(h, src2, dst2, et, n)
        h = _sage_mm(s, deg, h, q["wl"], q["wr"], q["bl"], act)
    s, _ = _pool_sum_max(h)
    return s / n


def kernel(module_x, module_edge_index, dispatch_x, dispatch_edge_index,
           type_x, type_edge_index, call_x, call_edge_index, params):
    hm = _module_enc(params["module"], module_x, module_edge_index)
    hd = _dispatch_enc(params["dispatch"], dispatch_x, dispatch_edge_index)
    ht = _type_enc(params["type"], type_x, type_edge_index)
    hc = _module_enc(params["call"], call_x, call_edge_index)
    h = jnp.concatenate([hm, hd, ht, hc], axis=1)
    f = params["fusion"]
    h = _mm(h, f["l1"]["w"], f["l1"]["b"], act="relu")
    return _mm(h, f["l2"]["w"], f["l2"]["b"])


# speedup vs baseline: 1.0930x; 1.0930x over previous
import functools

import jax
import jax.numpy as jnp
import numpy as np
from jax import lax
from jax.experimental import pallas as pl
from jax.experimental.pallas import tpu as pltpu
from jax.experimental.pallas import tpu_sc as plsc

_NC, _NS, _L = 2, 16, 16
_W = 2048
_SPMEM_BYTES = 6_500_000


def _prep_edges(ei):
    e = ei.shape[1]
    et = -(-e // (_NS * _W)) * _W
    e_pad = et * _NS
    pad = e_pad - e
    pad_src = (jnp.arange(pad, dtype=jnp.int32) % 256)
    src = jnp.concatenate([ei[0].astype(jnp.int32), pad_src])
    dst = jnp.concatenate([ei[1].astype(jnp.int32),
                           jnp.full((pad,), 1 << 29, jnp.int32)])
    return src.reshape(-1, _L), dst.reshape(-1, _L), et


def _chunking(n_out, d, tile_words=12000):
    budget_words = 1_950_000 - 16 * tile_words
    rmax = max(128, (budget_words // d - 64) // 128 * 128)
    n_chunks = -(-n_out // rmax)
    if n_chunks % 2:
        n_chunks += 1
    r = -(-n_out // (n_chunks * 128)) * 128
    return r, n_chunks


def _seg_sum_sc(x, src2, dst2, et, n_out):
    n_src, d = x.shape
    fb = 32 if d >= 128 else 64
    cap = _W + 160
    tile_words = 2 * (fb * d + 2 * fb) + 2 * 2048 + 2 * cap + 3000
    r, n_chunks = _chunking(n_out, d, tile_words)
    n_pad = r * n_chunks
    nw = et // _W
    trows = et // _L
    wrows = _W // _L
    nzb = (r + 64) // 64
    ncb = r // 64
    mesh = plsc.VectorSubcoreMesh(core_axis_name="c", subcore_axis_name="s",
                                  num_cores=_NC)

    @functools.partial(
        pl.kernel, mesh=mesh,
        compiler_params=pltpu.CompilerParams(needs_layout_passes=False, use_tc_tiling_on_sc=False),
        out_type=jax.ShapeDtypeStruct((n_pad, d), jnp.float32),
        scratch_types=[
            pltpu.VMEM((wrows, _L), jnp.int32),
            pltpu.VMEM((wrows, _L), jnp.int32),
            pltpu.VMEM((cap,), jnp.int32),
            pltpu.VMEM((cap,), jnp.int32),
            pltpu.VMEM((fb,), jnp.int32),
            pltpu.VMEM((fb,), jnp.int32),
            pltpu.VMEM((fb, d), jnp.float32),
            pltpu.VMEM((fb,), jnp.int32),
            pltpu.VMEM((fb,), jnp.int32),
            pltpu.VMEM((fb, d), jnp.float32),
            pltpu.VMEM_SHARED((r + 64, d), jnp.float32),
            pltpu.SemaphoreType.DMA,
            pltpu.SemaphoreType.DMA,
            pltpu.SemaphoreType.DMA,
            pltpu.SemaphoreType.DMA,
        ],
    )
    def k(x_hbm, src_hbm, dst_hbm, zeros_hbm, out_hbm,
          srcw, dstw, sstage, dstage, g_idx_a, sc_idx_a, rows_a,
          g_idx_b, sc_idx_b, rows_b, acc, sem1, sem2, sem3, sem4):
        bufs = ((g_idx_a, sc_idx_a, rows_a, sem1, sem3),
                (g_idx_b, sc_idx_b, rows_b, sem2, sem4))
        c = lax.axis_index("c")
        s = lax.axis_index("s")
        lane = lax.iota(jnp.int32, _L)
        pad_src = (s * _L + lane) % n_src

        for ci in range(n_chunks // 2):
            chunk = c * (n_chunks // 2) + ci
            lo = chunk * r
            lo_v = jnp.full((_L,), 0, jnp.int32) + lo
            hi_v = lo_v + r

            for jj in range((nzb + _NS - 1) // _NS):
                j = jj * _NS + s

                @pl.when(j < nzb)
                def _zero():
                    pltpu.sync_copy(zeros_hbm, acc.at[pl.ds(j * 64, 64), :])
            plsc.subcore_barrier()

            def window(w, _):
                wr0 = s * trows + w * wrows
                pltpu.sync_copy(src_hbm.at[pl.ds(wr0, wrows), :], srcw)
                pltpu.sync_copy(dst_hbm.at[pl.ds(wr0, wrows), :], dstw)

                def scan(b, cnt):
                    dvec = dstw[b, :]
                    svec = srcw[b, :]
                    m = (dvec >= lo_v) & (dvec < hi_v)
                    plsc.store_compressed(sstage.at[pl.ds(cnt, _L)], svec, mask=m)
                    plsc.store_compressed(dstage.at[pl.ds(cnt, _L)],
                                          dvec - lo_v, mask=m)
                    return cnt + plsc.all_reduce_population_count(m)[0]
                cnt = lax.fori_loop(0, wrows, scan, jnp.int32(0))

                trash = jnp.full((_L,), r, jnp.int32)
                for kk in range(fb // _L):
                    sstage[pl.ds(cnt + kk * _L, _L)] = pad_src
                    dstage[pl.ds(cnt + kk * _L, _L)] = trash

                nb = (cnt + fb - 1) // fb

                def drain_scatter(half):
                    gi, sci, rws, gsem, ssem = bufs[half]
                    pltpu.make_async_copy(rws, acc.at[sci], ssem).wait()

                def fetch(j, half):
                    gi, sci, rws, gsem, ssem = bufs[half]

                    @pl.when((j >= 2) & (j < nb))
                    def _dr():
                        drain_scatter(half)

                    @pl.when(j < nb)
                    def _():
                        for kk in range(fb // _L):
                            gi[pl.ds(kk * _L, _L)] = (
                                sstage[pl.ds(j * fb + kk * _L, _L)])
                            sci[pl.ds(kk * _L, _L)] = (
                                dstage[pl.ds(j * fb + kk * _L, _L)])
                        pltpu.async_copy(x_hbm.at[gi], rws, gsem)

                def process(j, half):
                    gi, sci, rws, gsem, ssem = bufs[half]

                    @pl.when(j < nb)
                    def _():
                        pltpu.make_async_copy(x_hbm.at[gi], rws, gsem).wait()
                        pltpu.async_copy(rws, acc.at[sci], ssem, add=True)

                fetch(0, 0)
                fetch(1, 1)

                def piped(jj, _):
                    j0 = jj * 2
                    process(j0, 0)
                    process(j0 + 1, 1)
                    fetch(j0 + 2, 0)
                    fetch(j0 + 3, 1)
                    return 0
                lax.fori_loop(0, (nb + 1) // 2, piped, 0)

                @pl.when(nb >= 1)
                def _d0():
                    drain_scatter(0)

                @pl.when(nb >= 2)
                def _d1():
                    drain_scatter(1)
                return 0
            lax.fori_loop(0, nw, window, 0)
            plsc.subcore_barrier()

            for jj in range((ncb + _NS - 1) // _NS):
                j = jj * _NS + s

                @pl.when(j < ncb)
                def _out():
                    pltpu.sync_copy(acc.at[pl.ds(j * 64, 64), :],
                                    out_hbm.at[pl.ds(lo + j * 64, 64), :])
            plsc.subcore_barrier()

    zeros = jnp.zeros((64, d), jnp.float32)
    return k(x, src2, dst2, zeros)


def _degree_sc(dst2, et, n_out):
    r, n_chunks = _chunking(n_out, 8)
    n_pad = r * n_chunks
    nw = et // _W
    trows = et // _L
    wrows = _W // _L
    nzb = (r + 64) // 64
    ncb = r // 64
    mesh = plsc.VectorSubcoreMesh(core_axis_name="c", subcore_axis_name="s",
                                  num_cores=_NC)

    @functools.partial(
        pl.kernel, mesh=mesh,
        compiler_params=pltpu.CompilerParams(needs_layout_passes=False, use_tc_tiling_on_sc=False),
        out_type=jax.ShapeDtypeStruct((n_pad, 8), jnp.float32),
        scratch_types=[
            pltpu.VMEM((wrows, _L), jnp.int32),
            pltpu.VMEM((128,), jnp.int32),
            pltpu.VMEM((128, 8), jnp.float32),
            pltpu.VMEM_SHARED((r + 64, 8), jnp.float32),
        ],
    )
    def k(dst_hbm, ones_hbm, zeros_hbm, out_hbm, dstw, sc_idx, ones, acc):
        c = lax.axis_index("c")
        s = lax.axis_index("s")
        pltpu.sync_copy(ones_hbm, ones)

        for ci in range(n_chunks // 2):
            chunk = c * (n_chunks // 2) + ci
            lo = chunk * r
            lo_v = jnp.full((_L,), 0, jnp.int32) + lo
            hi_v = lo_v + r

            for jj in range((nzb + _NS - 1) // _NS):
                j = jj * _NS + s

                @pl.when(j < nzb)
                def _zero():
                    pltpu.sync_copy(zeros_hbm, acc.at[pl.ds(j * 64, 64), :])
            plsc.subcore_barrier()

            def window(w, _):
                wr0 = s * trows + w * wrows
                pltpu.sync_copy(dst_hbm.at[pl.ds(wr0, wrows), :], dstw)

                def batch(j, _):
                    trash = jnp.full((_L,), r, jnp.int32)
                    for kk in range(8):
                        dvec = dstw[j * 8 + kk, :]
                        m = (dvec >= lo_v) & (dvec < hi_v)
                        sc_idx[pl.ds(kk * _L, _L)] = jnp.where(
                            m, dvec - lo_v, trash)
                    pltpu.sync_copy(ones, acc.at[sc_idx], add=True)
                    return 0
                lax.fori_loop(0, wrows // 8, batch, 0)
                return 0
            lax.fori_loop(0, nw, window, 0)
            plsc.subcore_barrier()

            for jj in range((ncb + _NS - 1) // _NS):
                j = jj * _NS + s

                @pl.when(j < ncb)
                def _out():
                    pltpu.sync_copy(acc.at[pl.ds(j * 64, 64), :],
                                    out_hbm.at[pl.ds(lo + j * 64, 64), :])
            plsc.subcore_barrier()

    ones = jnp.ones((128, 8), jnp.float32)
    zeros = jnp.zeros((64, 8), jnp.float32)
    return k(dst2, ones, zeros)


def _edge_softmax_sc(a, src2, dst2, et, n_out, cvec):
    n = a.shape[0]
    heads = 4
    r, n_chunks = _chunking(n_out, 8)
    n_pad = r * n_chunks
    nw = et // _W
    trows = et // _L
    wrows = _W // _L
    e_pad = et * _NS
    nzb = (r + 64) // 64
    ncb = r // 64
    mesh = plsc.VectorSubcoreMesh(core_axis_name="c", subcore_axis_name="s",
                                  num_cores=_NC)

    @functools.partial(
        pl.kernel, mesh=mesh,
        compiler_params=pltpu.CompilerParams(needs_layout_passes=False, use_tc_tiling_on_sc=False),
        out_type=[jax.ShapeDtypeStruct((e_pad, 8), jnp.float32),
                  jax.ShapeDtypeStruct((n_pad, 8), jnp.float32)],
        scratch_types=[
            pltpu.VMEM((wrows, _L), jnp.int32),
            pltpu.VMEM((wrows, _L), jnp.int32),
            pltpu.VMEM((128,), jnp.int32),
            pltpu.VMEM((128,), jnp.int32),
            pltpu.VMEM((128,), jnp.int32),
            pltpu.VMEM((128, 8), jnp.float32),
            pltpu.VMEM((128, 8), jnp.float32),
            pltpu.VMEM((128, 8), jnp.float32),
            pltpu.VMEM((_L,), jnp.float32),
            pltpu.VMEM_SHARED((r + 64, 8), jnp.float32),
            pltpu.SemaphoreType.DMA,
            pltpu.SemaphoreType.DMA,
        ],
    )
    def k(a_hbm, src_hbm, dst_hbm, c_hbm, zeros_hbm, ex_hbm, den_hbm,
          srcw, dstw, g1_idx, g2_idx, sc_idx, arows_s, arows_d, exw, cbuf,
          acc, sem1, sem2):
        c = lax.axis_index("c")
        s = lax.axis_index("s")
        lane = lax.iota(jnp.int32, _L)
        row4b = lane // heads
        colS = lane % heads
        pltpu.sync_copy(c_hbm, cbuf)
        cv = cbuf[...]

        chunk = c
        lo = chunk * r
        lo_v = jnp.full((_L,), 0, jnp.int32) + lo
        hi_v = lo_v + r
        trash = jnp.full((_L,), r, jnp.int32)

        for jj in range((nzb + _NS - 1) // _NS):
            j = jj * _NS + s

            @pl.when(j < nzb)
            def _zero():
                pltpu.sync_copy(zeros_hbm, acc.at[pl.ds(j * 64, 64), :])
        plsc.subcore_barrier()

        def window(w, _):
            wr0 = s * trows + w * wrows
            pltpu.sync_copy(src_hbm.at[pl.ds(wr0, wrows), :], srcw)
            pltpu.sync_copy(dst_hbm.at[pl.ds(wr0, wrows), :], dstw)

            def batch(j, _):
                nmax = jnp.full((_L,), n - 1, jnp.int32)
                for kk in range(8):
                    g1_idx[pl.ds(kk * _L, _L)] = jnp.minimum(
                        srcw[j * 8 + kk, :], nmax)
                    g2_idx[pl.ds(kk * _L, _L)] = jnp.minimum(
                        dstw[j * 8 + kk, :], nmax)
                cp1 = pltpu.async_copy(a_hbm.at[g1_idx], arows_s, sem1)
                cp2 = pltpu.async_copy(a_hbm.at[g2_idx], arows_d, sem2)
                cp1.wait()
                cp2.wait()

                def grp(b2, _):
                    row = b2 * 4 + row4b
                    vs = plsc.load_gather(arows_s, [row, colS])
                    vd = plsc.load_gather(arows_d, [row, colS + heads])
                    v = vs + vd
                    v = jnp.where(v >= 0.0, v, 0.2 * v) - cv
                    plsc.store_scatter(exw, [row, colS], jnp.exp(v))
                    return 0
                lax.fori_loop(0, 32, grp, 0)

                for kk in range(8):
                    dvec = dstw[j * 8 + kk, :]
                    m = (dvec >= lo_v) & (dvec < hi_v)
                    sc_idx[pl.ds(kk * _L, _L)] = jnp.where(
                        m, dvec - lo_v, trash)
                pltpu.sync_copy(exw, acc.at[sc_idx], add=True)

                @pl.when(c == 0)
                def _wr_ex():
                    pltpu.sync_copy(
                        exw, ex_hbm.at[pl.ds(s * et + w * _W + j * 128,
                                             128), :])
                return 0
            lax.fori_loop(0, wrows // 8, batch, 0)
            return 0
        lax.fori_loop(0, nw, window, 0)
        plsc.subcore_barrier()

        for jj in range((ncb + _NS - 1) // _NS):
            j = jj * _NS + s

            @pl.when(j < ncb)
            def _out():
                pltpu.sync_copy(acc.at[pl.ds(j * 64, 64), :],
                                den_hbm.at[pl.ds(lo + j * 64, 64), :])
        plsc.subcore_barrier()

    zeros = jnp.zeros((64, 8), jnp.float32)
    return k(a, src2, dst2, cvec, zeros)


def _weighted_seg_sum_sc(xw, ex, src2, dst2, et, n_out):
    n_src, d = xw.shape
    heads = 4
    o = d // heads
    fb = 32
    cap = _W + 160
    tile_words = 2 * (fb * d + fb * 8 + 3 * fb) + 2 * 2048 + 3 * cap + 3000
    r, n_chunks = _chunking(n_out, d, tile_words)
    n_pad = r * n_chunks
    nw = et // _W
    trows = et // _L
    wrows = _W // _L
    nzb = (r + 64) // 64
    ncb = r // 64
    mesh = plsc.VectorSubcoreMesh(core_axis_name="c", subcore_axis_name="s",
                                  num_cores=_NC)

    @functools.partial(
        pl.kernel, mesh=mesh,
        compiler_params=pltpu.CompilerParams(needs_layout_passes=False, use_tc_tiling_on_sc=False),
        out_type=jax.ShapeDtypeStruct((n_pad, d), jnp.float32),
        scratch_types=[
            pltpu.VMEM((wrows, _L), jnp.int32),
            pltpu.VMEM((wrows, _L), jnp.int32),
            pltpu.VMEM((wrows, _L), jnp.int32),
            pltpu.VMEM((wrows, _L), jnp.int32),
            pltpu.VMEM((cap,), jnp.int32),
            pltpu.VMEM((cap,), jnp.int32),
            pltpu.VMEM((cap,), jnp.int32),
            pltpu.VMEM((fb,), jnp.int32),
            pltpu.VMEM((fb,), jnp.int32),
            pltpu.VMEM((fb,), jnp.int32),
            pltpu.VMEM((fb, d), jnp.float32),
            pltpu.VMEM((fb, 8), jnp.float32),
            pltpu.VMEM((fb,), jnp.int32),
            pltpu.VMEM((fb,), jnp.int32),
            pltpu.VMEM((fb,), jnp.int32),
            pltpu.VMEM((fb, d), jnp.float32),
            pltpu.VMEM((fb, 8), jnp.float32),
            pltpu.VMEM_SHARED((r + 64, d), jnp.float32),
            pltpu.SemaphoreType.DMA,
            pltpu.SemaphoreType.DMA,
            pltpu.SemaphoreType.DMA,
            pltpu.SemaphoreType.DMA,
            pltpu.SemaphoreType.DMA,
            pltpu.SemaphoreType.DMA,
        ],
    )
    def k(xw_hbm, ex_hbm, src_hbm, dst_hbm, zeros_hbm, out_hbm,
          srcw_a, dstw_a, srcw_b, dstw_b, sstage, dstage, estage,
          g_idx_a, e_idx_a, sc_idx_a, rows_a, exr_a,
          g_idx_b, e_idx_b, sc_idx_b, rows_b, exr_b,
          acc, sem1, sem2, sem3, sem4, sem5, sem6):
        bufs = ((g_idx_a, e_idx_a, sc_idx_a, rows_a, exr_a, sem1, sem3),
                (g_idx_b, e_idx_b, sc_idx_b, rows_b, exr_b, sem2, sem4))
        wbufs = ((srcw_a, dstw_a, sem5), (srcw_b, dstw_b, sem6))
        c = lax.axis_index("c")
        s = lax.axis_index("s")
        lane = lax.iota(jnp.int32, _L)
        pad_src = (s * _L + lane) % n_src
        pad_eid = s * _L + lane

        def chunk_body(ci, _carry):
            chunk = c * (n_chunks // 2) + ci
            lo = chunk * r
            lo_v = jnp.full((_L,), 0, jnp.int32) + lo
            hi_v = lo_v + r
            trash = jnp.full((_L,), r, jnp.int32)

            for jj in range((nzb + _NS - 1) // _NS):
                j = jj * _NS + s

                @pl.when(j < nzb)
                def _zero():
                    pltpu.sync_copy(zeros_hbm, acc.at[pl.ds(j * 64, 64), :])
            plsc.subcore_barrier()

            def wfetch(w, whalf):
                sw, dw, wsem = wbufs[whalf]

                @pl.when(w < nw)
                def _():
                    wr0 = s * trows + w * wrows
                    pltpu.async_copy(src_hbm.at[pl.ds(wr0, wrows), :],
                                     sw, wsem)
                    pltpu.async_copy(dst_hbm.at[pl.ds(wr0, wrows), :],
                                     dw, wsem)

            def window(w, whalf):
                sw, dw, wsem = wbufs[whalf]
                pltpu.make_async_copy(src_hbm.at[pl.ds(0, wrows), :],
                                      sw, wsem).wait()
                pltpu.make_async_copy(dst_hbm.at[pl.ds(0, wrows), :],
                                      dw, wsem).wait()
                ebase = s * et + w * _W

                def scan(b, cnt):
                    dvec = dw[b, :]
                    svec = sw[b, :]
                    evec = ebase + b * _L + lane
                    m = (dvec >= lo_v) & (dvec < hi_v)
                    plsc.store_compressed(sstage.at[pl.ds(cnt, _L)], svec, mask=m)
                    plsc.store_compressed(dstage.at[pl.ds(cnt, _L)],
                                          dvec - lo_v, mask=m)
                    plsc.store_compressed(estage.at[pl.ds(cnt, _L)], evec, mask=m)
                    return cnt + plsc.all_reduce_population_count(m)[0]
                cnt = lax.fori_loop(0, wrows, scan, jnp.int32(0))

                for kk in range(fb // _L):
                    sstage[pl.ds(cnt + kk * _L, _L)] = pad_src
                    dstage[pl.ds(cnt + kk * _L, _L)] = trash
                    estage[pl.ds(cnt + kk * _L, _L)] = pad_eid

                nb = (cnt + fb - 1) // fb

                def drain_scatter(half):
                    gi, eix, sci, rws, exv, gsem, ssem = bufs[half]
                    pltpu.make_async_copy(rws, acc.at[sci], ssem).wait()

                def fetch(j, half):
                    gi, eix, sci, rws, exv, gsem, ssem = bufs[half]

                    @pl.when((j >= 2) & (j < nb))
                    def _dr():
                        drain_scatter(half)

                    @pl.when(j < nb)
                    def _():
                        for kk in range(fb // _L):
                            gi[pl.ds(kk * _L, _L)] = (
                                sstage[pl.ds(j * fb + kk * _L, _L)])
                            sci[pl.ds(kk * _L, _L)] = (
                                dstage[pl.ds(j * fb + kk * _L, _L)])
                            eix[pl.ds(kk * _L, _L)] = (
                                estage[pl.ds(j * fb + kk * _L, _L)])
                        pltpu.async_copy(xw_hbm.at[gi], rws, gsem)
                        pltpu.async_copy(ex_hbm.at[eix], exv, gsem)

                def process(j, half):
                    gi, eix, sci, rws, exv, gsem, ssem = bufs[half]

                    @pl.when(j < nb)
                    def _():
                        pltpu.make_async_copy(xw_hbm.at[gi], rws, gsem).wait()
                        pltpu.make_async_copy(ex_hbm.at[eix], exv, gsem).wait()

                        def scale(rr, _):
                            for h in range(heads):
                                mult = plsc.load_gather(
                                    exv,
                                    [jnp.full((_L,), 0, jnp.int32) + rr,
                                     jnp.full((_L,), h, jnp.int32)])
                                for g in range(o // _L):
                                    c0 = h * o + g * _L
                                    rws[rr, pl.ds(c0, _L)] = (
                                        rws[rr, pl.ds(c0, _L)] * mult)
                            return 0
                        lax.fori_loop(0, fb, scale, 0)
                        pltpu.async_copy(rws, acc.at[sci], ssem, add=True)

                fetch(0, 0)
                fetch(1, 1)

                def piped(jj, _):
                    j0 = jj * 2
                    process(j0, 0)
                    process(j0 + 1, 1)
                    fetch(j0 + 2, 0)
                    fetch(j0 + 3, 1)
                    return 0
                lax.fori_loop(0, (nb + 1) // 2, piped, 0)

                @pl.when(nb >= 1)
                def _d0():
                    drain_scatter(0)

                @pl.when(nb >= 2)
                def _d1():
                    drain_scatter(1)
                return 0

            wfetch(0, 0)

            def wloop(ww, _):
                w0 = ww * 2
                wfetch(w0 + 1, 1)
                window(w0, 0)
                wfetch(w0 + 2, 0)

                @pl.when(w0 + 1 < nw)
                def _wo():
                    window(w0 + 1, 1)
                return 0
            lax.fori_loop(0, (nw + 1) // 2, wloop, 0)
            plsc.subcore_barrier()

            for jj in range((ncb + _NS - 1) // _NS):
                j = jj * _NS + s

                @pl.when(j < ncb)
                def _out():
                    pltpu.sync_copy(acc.at[pl.ds(j * 64, 64), :],
                                    out_hbm.at[pl.ds(lo + j * 64, 64), :])
            plsc.subcore_barrier()
            return 0

        lax.fori_loop(0, n_chunks // 2, chunk_body, 0)

    zeros = jnp.zeros((64, d), jnp.float32)
    return k(xw, ex, src2, dst2, zeros)


def _mm_body(x_ref, w_ref, b_ref, o_ref, *, act):
    h = jnp.dot(x_ref[...], w_ref[...], preferred_element_type=jnp.float32)
    h = h + b_ref[...]
    if act == "relu":
        h = jnp.maximum(h, 0.0)
    elif act == "elu":
        h = jnp.where(h > 0.0, h, jnp.exp(h) - 1.0)
    o_ref[...] = h


def _pick_bn(n):
    for bn in (2000, 1000, 500, 200, 100, 50, 25, 10, 8, 5, 4, 2, 1):
        if n % bn == 0:
            return bn
    return n


def _mm(x, w, b=None, act=None):
    n, k = x.shape
    o = w.shape[1]
    if b is None:
        b = jnp.zeros((o,), jnp.float32)
    bn = _pick_bn(n)
    return pl.pallas_call(
        functools.partial(_mm_body, act=act),
        grid=(n // bn,),
        in_specs=[
            pl.BlockSpec((bn, k), lambda i: (i, 0)),
            pl.BlockSpec((k, o), lambda i: (0, 0)),
            pl.BlockSpec((1, o), lambda i: (0, 0)),
        ],
        out_specs=pl.BlockSpec((bn, o), lambda i: (i, 0)),
        out_shape=jax.ShapeDtypeStruct((n, o), jnp.float32),
    )(x, w, b.reshape(1, o))


def _pool_body(x_ref, s_ref, m_ref):
    i = pl.program_id(0)
    blk = x_ref[...]
    bs = jnp.sum(blk, axis=0, keepdims=True)
    bm = jnp.max(blk, axis=0, keepdims=True)

    @pl.when(i == 0)
    def _init():
        s_ref[...] = bs
        m_ref[...] = bm

    @pl.when(i > 0)
    def _acc():
        s_ref[...] = s_ref[...] + bs
        m_ref[...] = jnp.maximum(m_ref[...], bm)


def _pool_sum_max(x):
    n, d = x.shape
    bn = _pick_bn(n)
    return pl.pallas_call(
        _pool_body,
        grid=(n // bn,),
        in_specs=[pl.BlockSpec((bn, d), lambda i: (i, 0))],
        out_specs=[pl.BlockSpec((1, d), lambda i: (0, 0)),
                   pl.BlockSpec((1, d), lambda i: (0, 0))],
        out_shape=[jax.ShapeDtypeStruct((1, d), jnp.float32),
                   jax.ShapeDtypeStruct((1, d), jnp.float32)],
    )(x)


def _seg_sum(vals, idx, n):
    return jax.ops.segment_sum(vals, idx, num_segments=n)


def _gin_mm_body(h_ref, agg_ref, eps_ref, w_ref, b_ref, o_ref):
    z = (1.0 + eps_ref[0, 0]) * h_ref[...] + agg_ref[...]
    z = jnp.dot(z, w_ref[...], preferred_element_type=jnp.float32)
    o_ref[...] = jnp.maximum(z + b_ref[...], 0.0)


def _gin_mm(h, agg_pad, eps, w, b):
    n, k = h.shape
    o = w.shape[1]
    bn = _pick_bn(n)
    return pl.pallas_call(
        _gin_mm_body,
        grid=(n // bn,),
        in_specs=[
            pl.BlockSpec((bn, k), lambda i: (i, 0)),
            pl.BlockSpec((bn, k), lambda i: (i, 0)),
            pl.BlockSpec((1, 1), lambda i: (0, 0), memory_space=pltpu.SMEM),
            pl.BlockSpec((k, o), lambda i: (0, 0)),
            pl.BlockSpec((1, o), lambda i: (0, 0)),
        ],
        out_specs=pl.BlockSpec((bn, o), lambda i: (i, 0)),
        out_shape=jax.ShapeDtypeStruct((n, o), jnp.float32),
    )(h, agg_pad, eps.reshape(1, 1), w, b.reshape(1, o))


def _sage_mm_body(s_ref, deg_ref, x_ref, wl_ref, wr_ref, b_ref, o_ref, *, act):
    inv = 1.0 / jnp.maximum(deg_ref[...][:, 0:1], 1.0)
    mean = s_ref[...] * inv
    z = (jnp.dot(mean, wl_ref[...], preferred_element_type=jnp.float32)
         + jnp.dot(x_ref[...], wr_ref[...], preferred_element_type=jnp.float32)
         + b_ref[...])
    if act == "relu":
        z = jnp.maximum(z, 0.0)
    o_ref[...] = z


def _sage_mm(s_pad, deg_pad, x, wl, wr, b, act):
    n, k = x.shape
    o = wl.shape[1]
    bn = _pick_bn(n)
    return pl.pallas_call(
        functools.partial(_sage_mm_body, act=act),
        grid=(n // bn,),
        in_specs=[
            pl.BlockSpec((bn, k), lambda i: (i, 0)),
            pl.BlockSpec((bn, 8), lambda i: (i, 0)),
            pl.BlockSpec((bn, k), lambda i: (i, 0)),
            pl.BlockSpec((k, o), lambda i: (0, 0)),
            pl.BlockSpec((k, o), lambda i: (0, 0)),
            pl.BlockSpec((1, o), lambda i: (0, 0)),
        ],
        out_specs=pl.BlockSpec((bn, o), lambda i: (i, 0)),
        out_shape=jax.ShapeDtypeStruct((n, o), jnp.float32),
    )(s_pad, deg_pad, x, wl, wr, b.reshape(1, o))


def _gat_fin_body(num_ref, xw_ref, a_ref, den_ref, c_ref, b_ref, o_ref,
                  *, heads, o, concat, act):
    bn = num_ref.shape[0]
    al = a_ref[...]
    e0 = al[:, :heads] + al[:, heads:2 * heads]
    e0 = jnp.where(e0 >= 0.0, e0, 0.2 * e0)
    el = jnp.exp(e0 - c_ref[0])
    den = den_ref[...][:, :heads] + el
    xw3 = xw_ref[...].reshape(bn, heads, o)
    num3 = num_ref[...].reshape(bn, heads, o) + el[:, :, None] * xw3
    out3 = num3 / den[:, :, None]
    if concat:
        z = out3.reshape(bn, heads * o) + b_ref[...]
    else:
        z = jnp.mean(out3, axis=1) + b_ref[...]
    if act == "elu":
        z = jnp.where(z > 0.0, z, jnp.exp(z) - 1.0)
    o_ref[...] = z


def _gat_finalize(num_pad, xw, a, den_pad, c, bias, concat, act):
    n, d = xw.shape
    heads = 4
    o = d // heads
    od = d if concat else o
    bn = _pick_bn(n)
    return pl.pallas_call(
        functools.partial(_gat_fin_body, heads=heads, o=o, concat=concat,
                          act=act),
        grid=(n // bn,),
        in_specs=[
            pl.BlockSpec((bn, d), lambda i: (i, 0)),
            pl.BlockSpec((bn, d), lambda i: (i, 0)),
            pl.BlockSpec((bn, 2 * heads), lambda i: (i, 0)),
            pl.BlockSpec((bn, 8), lambda i: (i, 0)),
            pl.BlockSpec(memory_space=pltpu.SMEM),
            pl.BlockSpec((1, od), lambda i: (0, 0)),
        ],
        out_specs=pl.BlockSpec((bn, od), lambda i: (i, 0)),
        out_shape=jax.ShapeDtypeStruct((n, od), jnp.float32),
    )(num_pad, xw, a, den_pad, c.reshape(1), bias.reshape(1, od))


def _gat_layer(p, h, src2, dst2, et, n, heads, o, concat, act):
    xw = _mm(h, p["W"])
    eye = jnp.eye(heads, dtype=jnp.float32)
    a_src = jnp.einsum("ho,hg->hog", p["att_src"], eye).reshape(heads * o, heads)
    a_dst = jnp.einsum("ho,hg->hog", p["att_dst"], eye).reshape(heads * o, heads)
    ab = jnp.concatenate([a_src, a_dst], axis=1)
    a = _mm(xw, ab)
    _, amax = _pool_sum_max(a)
    c = jnp.maximum(jnp.max(amax[0, :heads]) + jnp.max(amax[0, heads:]), 0.0)
    ex, den = _edge_softmax_sc(a, src2, dst2, et, n,
                               jnp.full((_L,), 1.0, jnp.float32) * c)
    num = _weighted_seg_sum_sc(xw, ex, src2, dst2, et, n)
    return _gat_finalize(num, xw, a, den, c, p["bias"], concat, act)


def _module_enc(p, x, ei):
    n = x.shape[0]
    src2, dst2, et = _prep_edges(ei)
    h = _mm(x, p["proj"]["w"], p["proj"]["b"], act="relu")
    h = _gat_layer(p["gat0"], h, src2, dst2, et, n, 4, 128, True, "elu")
    h = _gat_layer(p["gat1"], h, src2, dst2, et, n, 4, 128, True, "elu")
    h = _gat_layer(p["gat2"], h, src2, dst2, et, n, 4, 128, False, None)
    s, m = _pool_sum_max(h)
    hg = (s / n + m) / 2.0
    return _mm(hg, p["out"]["w"], p["out"]["b"])


def _dispatch_enc(p, x, ei):
    n = x.shape[0]
    src2, dst2, et = _prep_edges(ei)
    h = _mm(x, p["proj"]["w"], p["proj"]["b"], act="relu")
    for i in range(4):
        g = p["gin%d" % i]
        agg = _seg_sum_sc(h, src2, dst2, et, n)
        z = _gin_mm(h, agg, g["eps"], g["l1"]["w"], g["l1"]["b"])
        h = _mm(z, g["l2"]["w"], g["l2"]["b"], act="relu")
    s, _ = _pool_sum_max(h)
    hg = s / n
    hg = _mm(hg, p["out1"]["w"], p["out1"]["b"], act="relu")
    return _mm(hg, p["out2"]["w"], p["out2"]["b"])


def _type_enc(p, x, ei):
    n = x.shape[0]
    src2, dst2, et = _prep_edges(ei)
    deg = _degree_sc(dst2, et, n)
    h = x
    for name, act in (("sage0", "relu"), ("sage1", "relu"), ("sage2", None)):
        q = p[name]
        s = _seg_sum_sc(h, src2, dst2, et, n)
        h = _sage_mm(s, deg, h, q["wl"], q["wr"], q["bl"], act)
    s, _ = _pool_sum_max(h)
    return s / n


def kernel(module_x, module_edge_index, dispatch_x, dispatch_edge_index,
           type_x, type_edge_index, call_x, call_edge_index, params):
    hm = _module_enc(params["module"], module_x, module_edge_index)
    hd = _dispatch_enc(params["dispatch"], dispatch_x, dispatch_edge_index)
    ht = _type_enc(params["type"], type_x, type_edge_index)
    hc = _module_enc(params["call"], call_x, call_edge_index)
    h = jnp.concatenate([hm, hd, ht, hc], axis=1)
    f = params["fusion"]
    h = _mm(h, f["l1"]["w"], f["l1"]["b"], act="relu")
    return _mm(h, f["l2"]["w"], f["l2"]["b"])
